# Initial kernel scaffold; baseline (speedup 1.0000x reference)
import jax
import jax.numpy as jnp
from jax.experimental import pallas as pl


def kernel(scores, hard_load, soft_load):
    raise NotImplementedError("write your pallas kernel here")


# speedup vs baseline: 6.2705x; 6.2705x over previous
import functools

import jax
import jax.numpy as jnp
from jax import lax
from jax.experimental import pallas as pl
from jax.experimental.pallas import tpu as pltpu
from jax.experimental.pallas import tpu_sc as plsc

_E = 64
_K = 8
_DECAY = 0.99
_ROWS = 32768
_NW = 32
_RPW = _ROWS // _NW
_BLK = 16
_NBLK = _RPW // _BLK


def _sc_body(scores, out, blk, acc, obuf):
    cid = lax.axis_index("c")
    sid = lax.axis_index("s")
    wid = cid * 16 + sid
    iota = lax.iota(jnp.int32, 16)
    fzero = jnp.zeros((16,), jnp.float32)
    fone = jnp.ones((16,), jnp.float32)
    neg = jnp.full((16,), -jnp.inf, jnp.float32)

    for r in range(128):
        acc[r, :] = fzero

    ibase = iota * _E

    def block(g, carry):
        row0 = wid * _RPW + g * _BLK
        pltpu.sync_copy(scores.at[pl.ds(row0 * _E, _BLK * _E)], blk)

        t = [neg] * _K
        for e in range(_E):
            v = plsc.load_gather(blk, [ibase + e])
            for k in range(_K):
                nv = jnp.minimum(t[k], v)
                t[k] = jnp.maximum(t[k], v)
                v = nv
        m0 = t[0]
        denom = fone
        for k in range(1, _K):
            denom = denom + jnp.exp(t[k] - m0)
        inv = fone / denom
        thr = t[_K - 1]

        for e in range(_E):
            v = plsc.load_gather(blk, [ibase + e])
            sel = v >= thr
            w = jnp.where(sel, jnp.exp(v - m0) * inv, fzero)
            cnt = jnp.where(sel, fone, fzero)
            plsc.addupdate(acc.at[e], cnt)
            plsc.addupdate(acc.at[_E + e], w)
        return carry

    lax.fori_loop(0, _NBLK, block, 0)

    for j in range(8):
        ovec = fzero
        for l in range(16):
            s = jnp.sum(acc[j * 16 + l, :])
            ovec = jnp.where(iota == l, jnp.full((16,), s), ovec)
        obuf[pl.ds(j * 16, 16)] = ovec
    pltpu.sync_copy(obuf, out.at[wid])


_sc_partials = functools.partial(
    pl.kernel,
    out_type=jax.ShapeDtypeStruct((_NW, 128), jnp.float32),
    mesh=plsc.VectorSubcoreMesh(core_axis_name="c", subcore_axis_name="s"),
    compiler_params=pltpu.CompilerParams(needs_layout_passes=False),
    scratch_types=[
        pltpu.VMEM((_BLK * _E,), jnp.float32),
        pltpu.VMEM((128, 16), jnp.float32),
        pltpu.VMEM((128,), jnp.float32),
    ],
)(_sc_body)


def _combine_body(part_ref, prev_ref, out_ref):
    total = jnp.sum(part_ref[...], axis=0)
    out_ref[...] = prev_ref[...] * _DECAY + total * (1.0 - _DECAY)


_combine = pl.pallas_call(
    _combine_body,
    out_shape=jax.ShapeDtypeStruct((2, _E), jnp.float32),
    in_specs=[
        pl.BlockSpec(memory_space=pltpu.VMEM),
        pl.BlockSpec(memory_space=pltpu.VMEM),
    ],
    out_specs=pl.BlockSpec(memory_space=pltpu.VMEM),
)


@jax.jit
def kernel(scores, hard_load, soft_load):
    part = _sc_partials(scores.reshape(-1))
    prev = jnp.stack([hard_load, soft_load], axis=0)
    return _combine(part.reshape(_NW, 2, _E), prev)
